# baseline (speedup 1.0000x reference)
import functools

import jax
import jax.numpy as jnp
from jax import lax
from jax.experimental import pallas as pl
from jax.experimental.pallas import tpu as pltpu
from jax.experimental.pallas import tpu_sc as plsc

N = 10000
E = 320000
F = 128

NC = 2
NS = 16
NTILES = NC * NS

CHUNK = 32
NCHUNK = 320
GSZ = 32
NGROUP = NCHUNK // GSZ
NBUF = 8
LAG = NBUF - 2
EPT = CHUNK * NCHUNK
E_PAD = EPT * NTILES
N_PAD = 10240
RPT = N_PAD // NS

_BLK = 1000
_ER = E // 128 // (N // _BLK)
_EPR = E_PAD // 128 // (N // _BLK)
_PADR = _EPR - _ER


def _dotT(a, b):
    return lax.dot_general(a, b, (((1,), (1,)), ((), ())),
                           preferred_element_type=jnp.float32)


def _tc1_body(x_ref, w_ref, b_ref, wl_ref, h_ref, t_ref):
    h = jnp.maximum(_dotT(x_ref[...], w_ref[...]) + b_ref[...], 0.0)
    h_ref[...] = h
    t_ref[...] = _dotT(h, wl_ref[...])


def _tc2_body(acc_ref, h_ref, bl_ref, wr_ref, wl2_ref, h1_ref, t1_ref):
    a = acc_ref[0] + acc_ref[1]
    h1 = jnp.maximum(a + bl_ref[...] + _dotT(h_ref[...], wr_ref[...]), 0.0)
    h1_ref[...] = h1
    t1_ref[...] = _dotT(h1, wl2_ref[...])


def _tc3_body(acc_ref, h_ref, bl_ref, wr_ref, pw_ref, pb_ref, ow_ref,
              ob_ref, out_ref):
    a = acc_ref[0] + acc_ref[1]
    h2 = jnp.maximum(a + bl_ref[...] + _dotT(h_ref[...], wr_ref[...]), 0.0)
    h3 = jnp.maximum(_dotT(h2, pw_ref[...]) + pb_ref[...], 0.0)
    out_ref[...] = _dotT(h3, ow_ref[...]) + ob_ref[...]


def _row_spec():
    return pl.BlockSpec((_BLK, F), lambda i: (i, 0))


def _full_spec(shape):
    nd = len(shape)
    return pl.BlockSpec(shape, lambda i: (0,) * nd)


def _acc_spec():
    return pl.BlockSpec((NC, _BLK, F), lambda i: (0, i, 0))


def _tc1(x, w, b, wl):
    return pl.pallas_call(
        _tc1_body,
        grid=(N // _BLK,),
        in_specs=[_row_spec(), _full_spec((F, F)), _full_spec((1, F)),
                  _full_spec((F, F))],
        out_specs=[_row_spec(), _row_spec()],
        out_shape=[jax.ShapeDtypeStruct((N, F), jnp.float32)] * 2,
    )(x, w, b, wl)


def _tc2(acc, h, bl, wr, wl2):
    return pl.pallas_call(
        _tc2_body,
        grid=(N // _BLK,),
        in_specs=[_acc_spec(), _row_spec(), _full_spec((1, F)),
                  _full_spec((F, F)), _full_spec((F, F))],
        out_specs=[_row_spec(), _row_spec()],
        out_shape=[jax.ShapeDtypeStruct((N, F), jnp.float32)] * 2,
    )(acc, h, bl, wr, wl2)


def _tc3(acc, h, bl, wr, pw, pb, ow, ob):
    return pl.pallas_call(
        _tc3_body,
        grid=(N // _BLK,),
        in_specs=[_acc_spec(), _row_spec(), _full_spec((1, F)),
                  _full_spec((F, F)), _full_spec((F, F)), _full_spec((1, F)),
                  _full_spec((F, F)), _full_spec((1, F))],
        out_specs=_row_spec(),
        out_shape=jax.ShapeDtypeStruct((N, F), jnp.float32),
    )(acc, h, bl, wr, pw, pb, ow, ob)


def _seg_body(table_hbm, src_hbm, dst_hbm, out_hbm,
              src_v, dst_v, rows_v, acc_sh,
              *sems):
    c = lax.axis_index("c")
    s = lax.axis_index("s")
    tid = c * NS + s

    zvec = jnp.zeros((16,), jnp.float32)

    def zbody(i, carry):
        for k16 in range(F // 16):
            rows_v[i, pl.ds(k16 * 16, 16)] = zvec
        return carry

    zrows = NBUF * CHUNK
    lax.fori_loop(0, zrows, zbody, 0)
    base = s * RPT
    for r in range(RPT // zrows):
        pltpu.sync_copy(rows_v,
                        acc_sh.at[pl.ds(base + r * zrows, zrows)])
    pltpu.sync_copy(rows_v.at[pl.ds(0, RPT % zrows)],
                    acc_sh.at[pl.ds(base + RPT - RPT % zrows, RPT % zrows)])
    plsc.subcore_barrier()

    bufs = [rows_v.at[pl.ds(k * CHUNK, CHUNK)] for k in range(NBUF)]

    def gather(gb, j, k, sem):
        return pltpu.make_async_copy(table_hbm.at[src_v.at[gb, j]],
                                     bufs[k], sem)

    def scatter(gb, j, k, sem):
        return pltpu.async_copy(bufs[k], acc_sh.at[dst_v.at[gb, j]], sem,
                                add=True)

    def scatter_wait(gb, j, k, sem):
        pltpu.make_async_copy(bufs[k], acc_sh.at[dst_v.at[gb, j]],
                              sem).wait()

    sem_g = list(sems[:NBUF])
    sem_s = list(sems[NBUF:2 * NBUF])
    sem_idx = sems[2 * NBUF]

    pltpu.sync_copy(src_hbm.at[tid, 0], src_v.at[0])
    pltpu.sync_copy(dst_hbm.at[tid, 0], dst_v.at[0])

    for grp in range(NGROUP):
        gb = grp % 2
        nb = (grp + 1) % 2
        if grp + 1 < NGROUP:
            pltpu.make_async_copy(src_hbm.at[tid, grp + 1],
                                  src_v.at[nb], sem_idx).start()
            pltpu.make_async_copy(dst_hbm.at[tid, grp + 1],
                                  dst_v.at[nb], sem_idx).start()

        gather(gb, 0, 0, sem_g[0]).start()
        gather(gb, 1, 1, sem_g[1]).start()

        def body(qq, carry, gb=gb):
            for k in range(NBUF):
                j = NBUF * qq + k
                ka = (k + 2) % NBUF
                gather(gb, j, k, sem_g[k]).wait()
                scatter(gb, j, k, sem_s[k])

                @pl.when(j + 2 < GSZ)
                def _(j=j, k=k, ka=ka):
                    @pl.when(j >= LAG)
                    def _():
                        scatter_wait(gb, j - LAG, ka, sem_s[ka])

                    gather(gb, j + 2, ka, sem_g[ka]).start()

            return carry

        lax.fori_loop(0, GSZ // NBUF, body, 0)

        for j in range(GSZ - NBUF, GSZ):
            scatter_wait(gb, j, j % NBUF, sem_s[j % NBUF])

        if grp + 1 < NGROUP:
            pltpu.make_async_copy(src_hbm.at[tid, grp + 1],
                                  src_v.at[nb], sem_idx).wait()
            pltpu.make_async_copy(dst_hbm.at[tid, grp + 1],
                                  dst_v.at[nb], sem_idx).wait()

    plsc.subcore_barrier()
    pltpu.sync_copy(acc_sh.at[pl.ds(base, RPT)],
                    out_hbm.at[c, pl.ds(base, RPT)])


@functools.cache
def _get_seg_sum():
    return functools.partial(
        pl.kernel,
        out_type=jax.ShapeDtypeStruct((NC, N_PAD, F), jnp.float32),
        mesh=plsc.VectorSubcoreMesh(core_axis_name="c", subcore_axis_name="s",
                                    num_cores=NC, num_subcores=NS),
        scratch_types=[
            pltpu.VMEM((2, GSZ, CHUNK), jnp.int32),
            pltpu.VMEM((2, GSZ, CHUNK), jnp.int32),
            pltpu.VMEM((NBUF * CHUNK, F), jnp.float32),
            pltpu.VMEM_SHARED((N_PAD, F), jnp.float32),
        ] + [pltpu.SemaphoreType.DMA] * (2 * NBUF + 1),
    )(_seg_body)


def _seg_sum(table, srcg, dstg):
    return _get_seg_sum()(table, srcg, dstg)


def kernel(x, edge_index, pre_W, pre_b, s1_Wl, s1_bl, s1_Wr,
           s2_Wl, s2_bl, s2_Wr, post_W, post_b, out_W, out_b):
    src = edge_index[0]
    dst = edge_index[1]
    pad = E_PAD - E
    pad_iota = jnp.arange(pad, dtype=jnp.int32)
    srcg = jnp.concatenate([src, pad_iota % N]
                           ).reshape(NTILES, NGROUP, GSZ, CHUNK)
    dstg = jnp.concatenate([dst, N + pad_iota % (N_PAD - N)]
                           ).reshape(NTILES, NGROUP, GSZ, CHUNK)

    pre_b2 = pre_b.reshape(1, F)
    s1_bl2 = s1_bl.reshape(1, F)
    s2_bl2 = s2_bl.reshape(1, F)
    post_b2 = post_b.reshape(1, F)
    out_b2 = out_b.reshape(1, F)

    h0, t0 = _tc1(x, pre_W, pre_b2, s1_Wl)
    acc1 = _seg_sum(t0, srcg, dstg)
    h1, t1 = _tc2(acc1, h0, s1_bl2, s1_Wr, s2_Wl)
    acc2 = _seg_sum(t1, srcg, dstg)
    return _tc3(acc2, h1, s2_bl2, s2_Wr, post_W, post_b2, out_W, out_b2)


# speedup vs baseline: 1.2336x; 1.2336x over previous
import functools

import jax
import jax.numpy as jnp
from jax import lax
from jax.experimental import pallas as pl
from jax.experimental.pallas import tpu as pltpu
from jax.experimental.pallas import tpu_sc as plsc

N = 10000
E = 320000
F = 128

NC = 2
NS = 16
NTILES = NC * NS

CHUNK = 64
NCHUNK = 160
GSZ = 32
NGROUP = NCHUNK // GSZ
NBUF = 4
LAG = NBUF - 2
EPT = CHUNK * NCHUNK
E_PAD = EPT * NTILES
N_PAD = 10240
RPT = N_PAD // NS

_BLK = 1000
_ER = E // 128 // (N // _BLK)
_EPR = E_PAD // 128 // (N // _BLK)
_PADR = _EPR - _ER


def _dotT(a, b):
    return lax.dot_general(a, b, (((1,), (1,)), ((), ())),
                           preferred_element_type=jnp.float32)


def _tc1_body(x_ref, w_ref, b_ref, wl_ref, h_ref, t_ref):
    h = jnp.maximum(_dotT(x_ref[...], w_ref[...]) + b_ref[...], 0.0)
    h_ref[...] = h
    t_ref[...] = _dotT(h, wl_ref[...])


def _tc2_body(acc_ref, h_ref, bl_ref, wr_ref, wl2_ref, h1_ref, t1_ref):
    a = acc_ref[0] + acc_ref[1]
    h1 = jnp.maximum(a + bl_ref[...] + _dotT(h_ref[...], wr_ref[...]), 0.0)
    h1_ref[...] = h1
    t1_ref[...] = _dotT(h1, wl2_ref[...])


def _tc3_body(acc_ref, h_ref, bl_ref, wr_ref, pw_ref, pb_ref, ow_ref,
              ob_ref, out_ref):
    a = acc_ref[0] + acc_ref[1]
    h2 = jnp.maximum(a + bl_ref[...] + _dotT(h_ref[...], wr_ref[...]), 0.0)
    h3 = jnp.maximum(_dotT(h2, pw_ref[...]) + pb_ref[...], 0.0)
    out_ref[...] = _dotT(h3, ow_ref[...]) + ob_ref[...]


def _row_spec():
    return pl.BlockSpec((_BLK, F), lambda i: (i, 0))


def _full_spec(shape):
    nd = len(shape)
    return pl.BlockSpec(shape, lambda i: (0,) * nd)


def _acc_spec():
    return pl.BlockSpec((NC, _BLK, F), lambda i: (0, i, 0))


def _tc1(x, w, b, wl):
    return pl.pallas_call(
        _tc1_body,
        grid=(N // _BLK,),
        in_specs=[_row_spec(), _full_spec((F, F)), _full_spec((1, F)),
                  _full_spec((F, F))],
        out_specs=[_row_spec(), _row_spec()],
        out_shape=[jax.ShapeDtypeStruct((N, F), jnp.float32)] * 2,
    )(x, w, b, wl)


def _tc2(acc, h, bl, wr, wl2):
    return pl.pallas_call(
        _tc2_body,
        grid=(N // _BLK,),
        in_specs=[_acc_spec(), _row_spec(), _full_spec((1, F)),
                  _full_spec((F, F)), _full_spec((F, F))],
        out_specs=[_row_spec(), _row_spec()],
        out_shape=[jax.ShapeDtypeStruct((N, F), jnp.float32)] * 2,
    )(acc, h, bl, wr, wl2)


def _tc3(acc, h, bl, wr, pw, pb, ow, ob):
    return pl.pallas_call(
        _tc3_body,
        grid=(N // _BLK,),
        in_specs=[_acc_spec(), _row_spec(), _full_spec((1, F)),
                  _full_spec((F, F)), _full_spec((F, F)), _full_spec((1, F)),
                  _full_spec((F, F)), _full_spec((1, F))],
        out_specs=_row_spec(),
        out_shape=jax.ShapeDtypeStruct((N, F), jnp.float32),
    )(acc, h, bl, wr, pw, pb, ow, ob)


def _seg_body(table_hbm, src_hbm, dst_hbm, out_hbm,
              src_v, dst_v, rows_v, acc_sh,
              *sems):
    c = lax.axis_index("c")
    s = lax.axis_index("s")
    tid = c * NS + s

    zvec = jnp.zeros((16,), jnp.float32)

    def zbody(i, carry):
        for k16 in range(F // 16):
            rows_v[i, pl.ds(k16 * 16, 16)] = zvec
        return carry

    zrows = NBUF * CHUNK
    lax.fori_loop(0, zrows, zbody, 0)
    base = s * RPT
    for r in range(RPT // zrows):
        pltpu.sync_copy(rows_v,
                        acc_sh.at[pl.ds(base + r * zrows, zrows)])
    pltpu.sync_copy(rows_v.at[pl.ds(0, RPT % zrows)],
                    acc_sh.at[pl.ds(base + RPT - RPT % zrows, RPT % zrows)])
    plsc.subcore_barrier()

    bufs = [rows_v.at[pl.ds(k * CHUNK, CHUNK)] for k in range(NBUF)]

    def gather(gb, j, k, sem):
        return pltpu.make_async_copy(table_hbm.at[src_v.at[gb, j]],
                                     bufs[k], sem)

    def scatter(gb, j, k, sem):
        return pltpu.async_copy(bufs[k], acc_sh.at[dst_v.at[gb, j]], sem,
                                add=True)

    def scatter_wait(gb, j, k, sem):
        pltpu.make_async_copy(bufs[k], acc_sh.at[dst_v.at[gb, j]],
                              sem).wait()

    sem_g = list(sems[:NBUF])
    sem_s = list(sems[NBUF:2 * NBUF])
    sem_idx = sems[2 * NBUF]

    pltpu.sync_copy(src_hbm.at[tid, 0], src_v.at[0])
    pltpu.sync_copy(dst_hbm.at[tid, 0], dst_v.at[0])

    for grp in range(NGROUP):
        gb = grp % 2
        nb = (grp + 1) % 2
        if grp + 1 < NGROUP:
            pltpu.make_async_copy(src_hbm.at[tid, grp + 1],
                                  src_v.at[nb], sem_idx).start()
            pltpu.make_async_copy(dst_hbm.at[tid, grp + 1],
                                  dst_v.at[nb], sem_idx).start()

        gather(gb, 0, 0, sem_g[0]).start()
        gather(gb, 1, 1, sem_g[1]).start()

        def body(qq, carry, gb=gb):
            for k in range(NBUF):
                j = NBUF * qq + k
                ka = (k + 2) % NBUF
                gather(gb, j, k, sem_g[k]).wait()
                scatter(gb, j, k, sem_s[k])

                @pl.when(j + 2 < GSZ)
                def _(j=j, k=k, ka=ka):
                    @pl.when(j >= LAG)
                    def _():
                        scatter_wait(gb, j - LAG, ka, sem_s[ka])

                    gather(gb, j + 2, ka, sem_g[ka]).start()

            return carry

        lax.fori_loop(0, GSZ // NBUF, body, 0)

        for j in range(GSZ - NBUF, GSZ):
            scatter_wait(gb, j, j % NBUF, sem_s[j % NBUF])

        if grp + 1 < NGROUP:
            pltpu.make_async_copy(src_hbm.at[tid, grp + 1],
                                  src_v.at[nb], sem_idx).wait()
            pltpu.make_async_copy(dst_hbm.at[tid, grp + 1],
                                  dst_v.at[nb], sem_idx).wait()

    plsc.subcore_barrier()
    pltpu.sync_copy(acc_sh.at[pl.ds(base, RPT)],
                    out_hbm.at[c, pl.ds(base, RPT)])


@functools.cache
def _get_seg_sum():
    return functools.partial(
        pl.kernel,
        out_type=jax.ShapeDtypeStruct((NC, N_PAD, F), jnp.float32),
        mesh=plsc.VectorSubcoreMesh(core_axis_name="c", subcore_axis_name="s",
                                    num_cores=NC, num_subcores=NS),
        scratch_types=[
            pltpu.VMEM((2, GSZ, CHUNK), jnp.int32),
            pltpu.VMEM((2, GSZ, CHUNK), jnp.int32),
            pltpu.VMEM((NBUF * CHUNK, F), jnp.float32),
            pltpu.VMEM_SHARED((N_PAD, F), jnp.float32),
        ] + [pltpu.SemaphoreType.DMA] * (2 * NBUF + 1),
    )(_seg_body)


def _seg_sum(table, srcg, dstg):
    return _get_seg_sum()(table, srcg, dstg)


def kernel(x, edge_index, pre_W, pre_b, s1_Wl, s1_bl, s1_Wr,
           s2_Wl, s2_bl, s2_Wr, post_W, post_b, out_W, out_b):
    src = edge_index[0]
    dst = edge_index[1]
    pad = E_PAD - E
    pad_iota = jnp.arange(pad, dtype=jnp.int32)
    srcg = jnp.concatenate([src, pad_iota % N]
                           ).reshape(NTILES, NGROUP, GSZ, CHUNK)
    dstg = jnp.concatenate([dst, N + pad_iota % (N_PAD - N)]
                           ).reshape(NTILES, NGROUP, GSZ, CHUNK)

    pre_b2 = pre_b.reshape(1, F)
    s1_bl2 = s1_bl.reshape(1, F)
    s2_bl2 = s2_bl.reshape(1, F)
    post_b2 = post_b.reshape(1, F)
    out_b2 = out_b.reshape(1, F)

    h0, t0 = _tc1(x, pre_W, pre_b2, s1_Wl)
    acc1 = _seg_sum(t0, srcg, dstg)
    h1, t1 = _tc2(acc1, h0, s1_bl2, s1_Wr, s2_Wl)
    acc2 = _seg_sum(t1, srcg, dstg)
    return _tc3(acc2, h1, s2_bl2, s2_Wr, post_W, post_b2, out_W, out_b2)
